# Initial kernel scaffold; baseline (speedup 1.0000x reference)
import jax
import jax.numpy as jnp
from jax.experimental import pallas as pl


def kernel(inputs, embedding):
    raise NotImplementedError("write your pallas kernel here")


# speedup vs baseline: 1.1354x; 1.1354x over previous
import jax
import jax.numpy as jnp
from jax.experimental import pallas as pl

_NUM_EMB = 1024
_DIM = 64
_ROWS = 16384
_BLK = 512
_GRID = _ROWS // _BLK
_COMMITMENT = 0.25


def _vq_body(x_ref, xsq_ref, embt_ref, emb_ref, esq_ref,
             enc_ref, q_ref, loss_ref):
    step = pl.program_id(0)
    x = x_ref[...]
    m = jnp.dot(x, embt_ref[...], preferred_element_type=jnp.float32)
    d = (xsq_ref[...] + esq_ref[...]) - 2.0 * m
    dmin = jnp.min(d, axis=1, keepdims=True)
    iota = jax.lax.broadcasted_iota(jnp.int32, d.shape, 1)
    idx = jnp.min(jnp.where(d == dmin, iota, _NUM_EMB), axis=1,
                  keepdims=True)
    enc = (iota == idx).astype(jnp.float32)
    enc_ref[...] = enc
    q = jnp.dot(enc, emb_ref[...], preferred_element_type=jnp.float32)
    q_ref[...] = x + (q - x)
    part = jnp.sum((q - x) ** 2).reshape(1, 1)

    @pl.when(step == 0)
    def _():
        loss_ref[...] = jnp.zeros_like(loss_ref)

    loss_ref[...] += part


def kernel(inputs, embedding):
    x = jnp.transpose(inputs, (0, 2, 3, 1))
    input_shape = x.shape
    flat = x.reshape(-1, _DIM).astype(jnp.float32)
    emb = embedding.astype(jnp.float32)
    xsq = jnp.sum(flat ** 2, axis=1, keepdims=True)
    esq = jnp.sum(emb ** 2, axis=1)[None, :]
    embt = emb.T

    enc, q, loss_sum = pl.pallas_call(
        _vq_body,
        grid=(_GRID,),
        in_specs=[
            pl.BlockSpec((_BLK, _DIM), lambda i: (i, 0)),
            pl.BlockSpec((_BLK, 1), lambda i: (i, 0)),
            pl.BlockSpec((_DIM, _NUM_EMB), lambda i: (0, 0)),
            pl.BlockSpec((_NUM_EMB, _DIM), lambda i: (0, 0)),
            pl.BlockSpec((1, _NUM_EMB), lambda i: (0, 0)),
        ],
        out_specs=[
            pl.BlockSpec((_BLK, _NUM_EMB), lambda i: (i, 0)),
            pl.BlockSpec((_BLK, _DIM), lambda i: (i, 0)),
            pl.BlockSpec((1, 1), lambda i: (0, 0)),
        ],
        out_shape=[
            jax.ShapeDtypeStruct((_ROWS, _NUM_EMB), jnp.float32),
            jax.ShapeDtypeStruct((_ROWS, _DIM), jnp.float32),
            jax.ShapeDtypeStruct((1, 1), jnp.float32),
        ],
    )(flat, xsq, embt, emb, esq)

    quantized = jnp.transpose(q.reshape(input_shape), (0, 3, 1, 2))
    loss = _COMMITMENT * (loss_sum[0, 0] / (_ROWS * _DIM))
    return (quantized, loss, enc)


# speedup vs baseline: 1.2885x; 1.1349x over previous
import jax
import jax.numpy as jnp
from jax.experimental import pallas as pl

_NUM_EMB = 1024
_DIM = 64
_HW = 1024
_IMGS = 16
_ROWS = _IMGS * _HW
_COMMITMENT = 0.25


def _vq_body(x_ref, xsq_ref, emb_ref, embt_ref, esq_ref,
             enc_ref, q_ref, loss_ref):
    step = pl.program_id(0)
    x = x_ref[0]
    mt = jax.lax.dot_general(emb_ref[...], x,
                             (((1,), (0,)), ((), ())),
                             preferred_element_type=jnp.float32)
    dt = (xsq_ref[0] + esq_ref[...]) - 2.0 * mt
    dmin = jnp.min(dt, axis=0, keepdims=True)
    iota = jax.lax.broadcasted_iota(jnp.int32, dt.shape, 0).astype(jnp.float32)
    idx = jnp.min(jnp.where(dt == dmin, iota, float(_NUM_EMB)),
                  axis=0, keepdims=True)
    onehot_t = jnp.where(iota == idx, 1.0, 0.0)
    enc_ref[...] = onehot_t.T
    q = jnp.dot(embt_ref[...], onehot_t,
                preferred_element_type=jnp.float32)
    q_ref[0] = x + (q - x)

    @pl.when(step == 0)
    def _():
        loss_ref[...] = jnp.zeros_like(loss_ref)

    loss_ref[...] += jnp.sum(dmin).reshape(1, 1)


def kernel(inputs, embedding):
    x_chw = inputs.astype(jnp.float32).reshape(_IMGS, _DIM, _HW)
    emb = embedding.astype(jnp.float32)
    flat = jnp.transpose(inputs, (0, 2, 3, 1)).reshape(-1, _DIM)
    flat = flat.astype(jnp.float32)
    xsq = jnp.sum(flat ** 2, axis=1).reshape(_IMGS, 1, _HW)
    esq = jnp.sum(emb ** 2, axis=1)[:, None]
    embt = emb.T

    enc, q, loss_sum = pl.pallas_call(
        _vq_body,
        grid=(_IMGS,),
        in_specs=[
            pl.BlockSpec((1, _DIM, _HW), lambda i: (i, 0, 0)),
            pl.BlockSpec((1, 1, _HW), lambda i: (i, 0, 0)),
            pl.BlockSpec((_NUM_EMB, _DIM), lambda i: (0, 0)),
            pl.BlockSpec((_DIM, _NUM_EMB), lambda i: (0, 0)),
            pl.BlockSpec((_NUM_EMB, 1), lambda i: (0, 0)),
        ],
        out_specs=[
            pl.BlockSpec((_HW, _NUM_EMB), lambda i: (i, 0)),
            pl.BlockSpec((1, _DIM, _HW), lambda i: (i, 0, 0)),
            pl.BlockSpec((1, 1), lambda i: (0, 0)),
        ],
        out_shape=[
            jax.ShapeDtypeStruct((_ROWS, _NUM_EMB), jnp.float32),
            jax.ShapeDtypeStruct((_IMGS, _DIM, _HW), jnp.float32),
            jax.ShapeDtypeStruct((1, 1), jnp.float32),
        ],
    )(x_chw, xsq, emb, embt, esq)

    quantized = q.reshape(inputs.shape)
    loss = _COMMITMENT * (loss_sum[0, 0] / (_ROWS * _DIM))
    return (quantized, loss, enc)
